# baseline (speedup 1.0000x reference)
import functools

import jax
import jax.numpy as jnp
from jax import lax
from jax.experimental import pallas as pl
from jax.experimental.pallas import tpu as pltpu
from jax.experimental.pallas import tpu_sc as plsc

_VOCAB = 100000
_EMB = 64
_BATCH = 16384
_HIST = 200

_NC = 2
_NS = 16
_NW = _NC * _NS
_ROWS_W = _BATCH // _NW
_CBLK = 128
_NCBLK = _ROWS_W // _CBLK
_LSPLIT = (96, 104)
_LOFF = (0, 96)
_GRP = 16
_NGRP = _CBLK // _GRP
_UNROLL = 2

_PBLK = 25600
_PGRID = (_VOCAB + _PBLK - 1) // _PBLK


def _proj_body(et_ref, w_ref, p_ref):
    p_ref[...] = jnp.sum(et_ref[...] * w_ref[...], axis=0)


def _proj(emb_t, fc_w_t):
    return pl.pallas_call(
        _proj_body,
        grid=(_PGRID,),
        in_specs=[
            pl.BlockSpec((_EMB, _PBLK), lambda i: (0, i)),
            pl.BlockSpec((_EMB, 1), lambda i: (0, 0)),
        ],
        out_specs=pl.BlockSpec((_PBLK,), lambda i: (i,)),
        out_shape=jax.ShapeDtypeStruct((_VOCAB,), jnp.float32),
    )(emb_t, fc_w_t)


_MESH = plsc.VectorSubcoreMesh(core_axis_name="c", subcore_axis_name="s")


@functools.partial(
    pl.kernel,
    out_type=jax.ShapeDtypeStruct((_BATCH,), jnp.float32),
    mesh=_MESH,
    compiler_params=pltpu.CompilerParams(needs_layout_passes=False),
    scratch_types=[
        pltpu.VMEM((_VOCAB,), jnp.float32),
        pltpu.VMEM((max(_LSPLIT), _CBLK), jnp.int32),
        pltpu.VMEM((max(_LSPLIT), _CBLK), jnp.int32),
        pltpu.VMEM((_ROWS_W,), jnp.float32),
        pltpu.VMEM((16,), jnp.float32),
        pltpu.SemaphoreType.DMA,
        pltpu.SemaphoreType.DMA,
        pltpu.SemaphoreType.DMA,
    ],
)
def _sc_pool(xt_hbm, p_hbm, b_hbm, out_hbm,
             p_v, xa, xb, out_v, b_v, sem_a, sem_b, sem_p):
    wid = lax.axis_index("s") * _NC + lax.axis_index("c")
    col0 = wid * _ROWS_W

    bufs = (xa, xb)
    sems = (sem_a, sem_b)
    copies = [None, None]

    def start_copy(step):
        cb, h = divmod(step, 2)
        return pltpu.async_copy(
            xt_hbm.at[pl.ds(_LOFF[h], _LSPLIT[h]),
                      pl.ds(col0 + cb * _CBLK, _CBLK)],
            bufs[step % 2].at[pl.ds(0, _LSPLIT[h]), :], sems[step % 2])

    copies[0] = start_copy(0)
    pltpu.sync_copy(b_hbm, b_v.at[pl.ds(0, 1)])
    pltpu.async_copy(p_hbm, p_v, sem_p).wait()
    bias = b_v[pl.ds(0, 16)][0]

    zf = jnp.zeros((16,), jnp.float32)
    zi = jnp.zeros((16,), jnp.int32)
    one = jnp.full((16,), 1, jnp.int32)

    nsteps = 2 * _NCBLK
    for cb in range(_NCBLK):
        accs = (zf, zi) * _NGRP
        for h in range(2):
            step = cb * 2 + h
            if step + 1 < nsteps:
                copies[(step + 1) % 2] = start_copy(step + 1)
            copies[step % 2].wait()
            xbuf = bufs[step % 2]

            def body(i, carry, xbuf=xbuf):
                l = i * _UNROLL
                for u in range(_UNROLL):
                    out = []
                    for g in range(_NGRP):
                        acc, cnt = carry[2 * g], carry[2 * g + 1]
                        xv = xbuf[l + u, pl.ds(g * _GRP, _GRP)]
                        pv = plsc.load_gather(p_v, [xv])
                        out.append(acc + pv)
                        out.append(cnt + jnp.minimum(xv, one))
                    carry = tuple(out)
                return carry

            accs = lax.fori_loop(0, _LSPLIT[h] // _UNROLL, body, accs)
        for g in range(_NGRP):
            acc, cnt = accs[2 * g], accs[2 * g + 1]
            res = acc / cnt.astype(jnp.float32) + bias
            out_v[pl.ds(cb * _CBLK + g * _GRP, _GRP)] = res

    pltpu.sync_copy(out_v, out_hbm.at[pl.ds(wid * _ROWS_W, _ROWS_W)])


def kernel(x, embedding, fc_w, fc_b):
    xt = x.astype(jnp.int32).T
    p = _proj(embedding.T, fc_w.T)
    out = _sc_pool(xt, p, fc_b)
    return out.reshape(_BATCH, 1)


# speedup vs baseline: 1.0133x; 1.0133x over previous
import functools

import jax
import jax.numpy as jnp
from jax import lax
from jax.experimental import pallas as pl
from jax.experimental.pallas import tpu as pltpu
from jax.experimental.pallas import tpu_sc as plsc

_VOCAB = 100000
_EMB = 64
_BATCH = 16384
_HIST = 200

_NC = 2
_NS = 16
_NW = _NC * _NS
_ROWS_W = _BATCH // _NW
_CBLK = 128
_NCBLK = _ROWS_W // _CBLK
_LSPLIT = (96, 104)
_LOFF = (0, 96)
_GRP = 16
_NGRP = _CBLK // _GRP
_UNROLL = 2

_PBLK = 25600
_PGRID = (_VOCAB + _PBLK - 1) // _PBLK


def _proj_body(et_ref, w_ref, p_ref):
    p_ref[...] = jnp.sum(et_ref[...] * w_ref[...], axis=0)


def _proj(emb_t, fc_w_t):
    return pl.pallas_call(
        _proj_body,
        grid=(_PGRID,),
        in_specs=[
            pl.BlockSpec((_EMB, _PBLK), lambda i: (0, i)),
            pl.BlockSpec((_EMB, 1), lambda i: (0, 0)),
        ],
        out_specs=pl.BlockSpec((_PBLK,), lambda i: (i,)),
        out_shape=jax.ShapeDtypeStruct((_VOCAB,), jnp.float32),
    )(emb_t, fc_w_t)


_MESH = plsc.VectorSubcoreMesh(core_axis_name="c", subcore_axis_name="s")


@functools.partial(
    pl.kernel,
    out_type=jax.ShapeDtypeStruct((_BATCH,), jnp.float32),
    mesh=_MESH,
    compiler_params=pltpu.CompilerParams(needs_layout_passes=False),
    scratch_types=[
        pltpu.VMEM((_VOCAB,), jnp.float32),
        pltpu.VMEM((max(_LSPLIT), _CBLK), jnp.int32),
        pltpu.VMEM((max(_LSPLIT), _CBLK), jnp.int32),
        pltpu.VMEM((_ROWS_W,), jnp.float32),
        pltpu.VMEM((16,), jnp.float32),
        pltpu.SemaphoreType.DMA,
        pltpu.SemaphoreType.DMA,
        pltpu.SemaphoreType.DMA,
    ],
)
def _sc_pool(xt_hbm, p_hbm, b_hbm, out_hbm,
             p_v, xa, xb, out_v, b_v, sem_a, sem_b, sem_p):
    wid = lax.axis_index("s") * _NC + lax.axis_index("c")
    col0 = wid * _ROWS_W

    def _copy_desc(cb, h, buf, sem, make):
        ctor = pltpu.make_async_copy if make else pltpu.async_copy
        return ctor(
            xt_hbm.at[pl.ds(_LOFF[h], _LSPLIT[h]),
                      pl.ds(col0 + cb * _CBLK, _CBLK)],
            buf.at[pl.ds(0, _LSPLIT[h]), :], sem)

    def start_copy(cb, h, buf, sem):
        return _copy_desc(cb, h, buf, sem, False)

    def wait_copy(cb, h, buf, sem):
        _copy_desc(cb, h, buf, sem, True).wait()

    start_copy(0, 0, xa, sem_a)
    pltpu.sync_copy(b_hbm, b_v.at[pl.ds(0, 1)])
    pltpu.async_copy(p_hbm, p_v, sem_p).wait()
    bias = b_v[pl.ds(0, 16)][0]

    zf = jnp.zeros((16,), jnp.float32)
    zi = jnp.zeros((16,), jnp.int32)
    one = jnp.full((16,), 1, jnp.int32)

    def half_sum(xbuf, h, accs):
        def body(i, carry, xbuf=xbuf):
            l = i * _UNROLL
            for u in range(_UNROLL):
                out = []
                for g in range(_NGRP):
                    acc, cnt = carry[2 * g], carry[2 * g + 1]
                    xv = xbuf[l + u, pl.ds(g * _GRP, _GRP)]
                    pv = plsc.load_gather(p_v, [xv])
                    out.append(acc + pv)
                    out.append(cnt + jnp.minimum(xv, one))
                carry = tuple(out)
            return carry

        return lax.fori_loop(0, _LSPLIT[h] // _UNROLL, body, accs)

    def cb_body(cb, dummy):
        start_copy(cb, 1, xb, sem_b)
        wait_copy(cb, 0, xa, sem_a)
        accs = half_sum(xa, 0, (zf, zi) * _NGRP)

        @pl.when(cb + 1 < _NCBLK)
        def _():
            start_copy(cb + 1, 0, xa, sem_a)

        wait_copy(cb, 1, xb, sem_b)
        accs = half_sum(xb, 1, accs)
        for g in range(_NGRP):
            acc, cnt = accs[2 * g], accs[2 * g + 1]
            res = acc / cnt.astype(jnp.float32) + bias
            out_v[pl.ds(cb * _CBLK + g * _GRP, _GRP)] = res
        return dummy

    lax.fori_loop(0, _NCBLK, cb_body, 0)

    pltpu.sync_copy(out_v, out_hbm.at[pl.ds(wid * _ROWS_W, _ROWS_W)])


def kernel(x, embedding, fc_w, fc_b):
    xt = x.astype(jnp.int32).T
    p = _proj(embedding.T, fc_w.T)
    out = _sc_pool(xt, p, fc_b)
    return out.reshape(_BATCH, 1)
